# baseline (speedup 1.0000x reference)
import jax
import jax.numpy as jnp
from jax.experimental import pallas as pl


def _fused_body(x_ref, wqt_ref, bq_ref, mkt_ref, mv_ref, g1xt_ref, g1rt_ref,
                g1b_ref, g2_ref, g2b_ref, bg_ref, bb_ref, og_ref, ob_ref,
                out_ref):
    tb = x_ref.shape[0]
    half = tb // 2
    for lo in (0, half):
        x = x_ref[lo:lo + half, :]
        xb = x.astype(jnp.bfloat16)

        q = jnp.dot(xb, wqt_ref[...], preferred_element_type=jnp.float32)
        q = q + bq_ref[...]

        sim = jnp.dot(q.astype(jnp.bfloat16), mkt_ref[...],
                      preferred_element_type=jnp.float32)
        sim = sim - jnp.max(sim, axis=-1, keepdims=True)
        e = jnp.exp(sim)
        attn = e / jnp.sum(e, axis=-1, keepdims=True)

        r = jnp.dot(attn.astype(jnp.bfloat16), mv_ref[...],
                    preferred_element_type=jnp.float32)

        gh = jnp.dot(xb, g1xt_ref[...], preferred_element_type=jnp.float32)
        gh = gh + jnp.dot(r.astype(jnp.bfloat16), g1rt_ref[...],
                          preferred_element_type=jnp.float32)
        gh = jnp.maximum(gh + g1b_ref[...], 0.0)

        logit = (jnp.sum(gh * g2_ref[...], axis=-1, keepdims=True)
                 + g2b_ref[0, 0])
        g = jax.nn.sigmoid(logit)

        u = (1.0 - g) * x + g * r

        mu = jnp.mean(u, axis=-1, keepdims=True)
        d = u - mu
        var = jnp.mean(d * d, axis=-1, keepdims=True)
        u = d * jax.lax.rsqrt(var + 1e-5) * bg_ref[...] + bb_ref[...]

        mu = jnp.mean(u, axis=-1, keepdims=True)
        d = u - mu
        var = jnp.mean(d * d, axis=-1, keepdims=True)
        out_ref[lo:lo + half, :] = (d * jax.lax.rsqrt(var + 1e-5)
                                    * og_ref[...] + ob_ref[...])


def kernel(hidden_states, layer_idx, memory_keys, memory_values, Wq, bq,
           Wk, bk, Wv, bv, G1, g1b, G2, g2b, bank_gamma, bank_beta,
           out_gamma, out_beta):
    B, S, H = hidden_states.shape
    M = memory_keys.shape[0]
    H2 = G1.shape[0]
    N = B * S
    TB = 256 if N % 256 == 0 else N

    x = hidden_states.reshape(N, H)
    wqt = Wq.T.astype(jnp.bfloat16)
    mkt = memory_keys.T.astype(jnp.bfloat16)
    mv = memory_values.astype(jnp.bfloat16)
    g1t = G1.T.astype(jnp.bfloat16)
    g1xt = g1t[:H]
    g1rt = g1t[H:]

    bq2 = bq.reshape(1, H)
    g1b2 = g1b.reshape(1, H2)
    g2row = G2.reshape(1, H2)
    g2b2 = g2b.reshape(1, 1)
    bg2 = bank_gamma.reshape(1, H)
    bb2 = bank_beta.reshape(1, H)
    og2 = out_gamma.reshape(1, H)
    ob2 = out_beta.reshape(1, H)

    fixed = lambda i: (0, 0)
    out = pl.pallas_call(
        _fused_body,
        grid=(N // TB,),
        in_specs=[
            pl.BlockSpec((TB, H), lambda i: (i, 0)),
            pl.BlockSpec((H, H), fixed),
            pl.BlockSpec((1, H), fixed),
            pl.BlockSpec((H, M), fixed),
            pl.BlockSpec((M, H), fixed),
            pl.BlockSpec((H, H2), fixed),
            pl.BlockSpec((H, H2), fixed),
            pl.BlockSpec((1, H2), fixed),
            pl.BlockSpec((1, H2), fixed),
            pl.BlockSpec((1, 1), fixed),
            pl.BlockSpec((1, H), fixed),
            pl.BlockSpec((1, H), fixed),
            pl.BlockSpec((1, H), fixed),
            pl.BlockSpec((1, H), fixed),
        ],
        out_specs=pl.BlockSpec((TB, H), lambda i: (i, 0)),
        out_shape=jax.ShapeDtypeStruct((N, H), jnp.float32),
    )(x, wqt, bq2, mkt, mv, g1xt, g1rt, g1b2, g2row, g2b2, bg2, bb2, og2, ob2)
    return out.reshape(B, S, H)


# speedup vs baseline: 1.0692x; 1.0692x over previous
import jax
import jax.numpy as jnp
from jax.experimental import pallas as pl


def _fused_body(x_ref, wqt_ref, bq_ref, mkt_ref, mv_ref, g1xt_ref, g1rt_ref,
                g1b_ref, g2_ref, g2b_ref, bg_ref, bb_ref, og_ref, ob_ref,
                out_ref):
    tb = x_ref.shape[0]
    half = tb // 2
    for lo in (0, half):
        x = x_ref[lo:lo + half, :]
        xb = x.astype(jnp.bfloat16)

        q = jnp.dot(xb, wqt_ref[...], preferred_element_type=jnp.float32)
        q = q + bq_ref[...]

        sim = jnp.dot(q.astype(jnp.bfloat16), mkt_ref[...],
                      preferred_element_type=jnp.float32)
        sim = sim - jnp.max(sim, axis=-1, keepdims=True)
        e = jnp.exp(sim)
        attn = e / jnp.sum(e, axis=-1, keepdims=True)

        r = jnp.dot(attn.astype(jnp.bfloat16), mv_ref[...],
                    preferred_element_type=jnp.float32)

        gh = jnp.dot(xb, g1xt_ref[...], preferred_element_type=jnp.float32)
        gh = gh + jnp.dot(r.astype(jnp.bfloat16), g1rt_ref[...],
                          preferred_element_type=jnp.float32)
        gh = jnp.maximum(gh + g1b_ref[...], 0.0)

        logit = (jnp.sum(gh * g2_ref[...], axis=-1, keepdims=True)
                 + g2b_ref[0, 0])
        g = jax.nn.sigmoid(logit)

        u = (1.0 - g) * x + g * r

        mu = jnp.mean(u, axis=-1, keepdims=True)
        d = u - mu
        var = jnp.mean(d * d, axis=-1, keepdims=True)
        u = d * jax.lax.rsqrt(var + 1e-5) * bg_ref[...] + bb_ref[...]

        mu = jnp.mean(u, axis=-1, keepdims=True)
        d = u - mu
        var = jnp.mean(d * d, axis=-1, keepdims=True)
        out_ref[lo:lo + half, :] = (d * jax.lax.rsqrt(var + 1e-5)
                                    * og_ref[...] + ob_ref[...])


def kernel(hidden_states, layer_idx, memory_keys, memory_values, Wq, bq,
           Wk, bk, Wv, bv, G1, g1b, G2, g2b, bank_gamma, bank_beta,
           out_gamma, out_beta):
    B, S, H = hidden_states.shape
    M = memory_keys.shape[0]
    H2 = G1.shape[0]
    N = B * S
    TB = 512 if N % 512 == 0 else N

    x = hidden_states.reshape(N, H)
    wqt = Wq.T.astype(jnp.bfloat16)
    mkt = memory_keys.T.astype(jnp.bfloat16)
    mv = memory_values.astype(jnp.bfloat16)
    g1t = G1.T.astype(jnp.bfloat16)
    g1xt = g1t[:H]
    g1rt = g1t[H:]

    bq2 = bq.reshape(1, H)
    g1b2 = g1b.reshape(1, H2)
    g2row = G2.reshape(1, H2)
    g2b2 = g2b.reshape(1, 1)
    bg2 = bank_gamma.reshape(1, H)
    bb2 = bank_beta.reshape(1, H)
    og2 = out_gamma.reshape(1, H)
    ob2 = out_beta.reshape(1, H)

    fixed = lambda i: (0, 0)
    out = pl.pallas_call(
        _fused_body,
        grid=(N // TB,),
        in_specs=[
            pl.BlockSpec((TB, H), lambda i: (i, 0)),
            pl.BlockSpec((H, H), fixed),
            pl.BlockSpec((1, H), fixed),
            pl.BlockSpec((H, M), fixed),
            pl.BlockSpec((M, H), fixed),
            pl.BlockSpec((H, H2), fixed),
            pl.BlockSpec((H, H2), fixed),
            pl.BlockSpec((1, H2), fixed),
            pl.BlockSpec((1, H2), fixed),
            pl.BlockSpec((1, 1), fixed),
            pl.BlockSpec((1, H), fixed),
            pl.BlockSpec((1, H), fixed),
            pl.BlockSpec((1, H), fixed),
            pl.BlockSpec((1, H), fixed),
        ],
        out_specs=pl.BlockSpec((TB, H), lambda i: (i, 0)),
        out_shape=jax.ShapeDtypeStruct((N, H), jnp.float32),
    )(x, wqt, bq2, mkt, mv, g1xt, g1rt, g1b2, g2row, g2b2, bg2, bb2, og2, ob2)
    return out.reshape(B, S, H)
